# baseline (speedup 1.0000x reference)
import numpy as np
import jax
import jax.numpy as jnp
from jax import lax
from jax.experimental import pallas as pl
from jax.experimental.pallas import tpu as pltpu
from jax.experimental.pallas import tpu_sc as plsc

F = 26
V = 1000
D = 128
B = 4096
FP = 32

NW = 32
RECW = B // NW
CRE = 4
CHV = CRE * F
CHP = CRE * FP
NCH = RECW // CRE
NBUF = 6
DIST = 4

_f = np.arange(CHP) % FP
_FOFF = np.where(_f < F, _f * V, 0).astype(np.int32)


def _fuse_body(tab_ref, bias_ref, out_ref):
    out_ref[...] = tab_ref[...] + bias_ref[...]


def _fuse(tables, bias):
    return pl.pallas_call(
        _fuse_body,
        grid=(F,),
        in_specs=[
            pl.BlockSpec((1, V, D), lambda f: (f, 0, 0)),
            pl.BlockSpec((1, 1, D), lambda f: (f, 0, 0)),
        ],
        out_specs=pl.BlockSpec((1, V, D), lambda f: (f, 0, 0)),
        out_shape=jax.ShapeDtypeStruct((F, V, D), jnp.float32),
    )(tables, bias.reshape(F, 1, D))


def _gather_body(x_hbm, foff_hbm, tab_hbm, out_hbm,
                 foff_v,
                 xs0, xs1, xs2, xs3, xs4, xs5,
                 ib0, ib1, ib2, ib3, ib4, ib5,
                 gb0, gb1, gb2, gb3, gb4, gb5,
                 gs0, gs1, gs2, gs3, gs4, gs5,
                 ss0, ss1, ss2, ss3, ss4, ss5):
    wid = lax.axis_index("s") * 2 + lax.axis_index("c")
    base = wid * RECW * F
    obase = wid * RECW * FP

    XS = (xs0, xs1, xs2, xs3, xs4, xs5)
    IB = (ib0, ib1, ib2, ib3, ib4, ib5)
    GB = (gb0, gb1, gb2, gb3, gb4, gb5)
    GS = (gs0, gs1, gs2, gs3, gs4, gs5)
    SS = (ss0, ss1, ss2, ss3, ss4, ss5)

    pltpu.sync_copy(foff_hbm, foff_v)

    def wait_store(q):
        pltpu.make_async_copy(GB[q], out_hbm.at[pl.ds(obase, CHP)], SS[q]).wait()

    def fetch(c, q, wait):
        if wait:
            wait_store(q)
        pltpu.sync_copy(x_hbm.at[pl.ds(obase + c * CHP, CHP)], XS[q])
        for g in range(CHP // 16):
            sl = pl.ds(g * 16, 16)
            IB[q][sl] = XS[q][sl] + foff_v[sl]
        pltpu.async_copy(tab_hbm.at[IB[q]], GB[q], GS[q])

    def body(c, p):
        pltpu.make_async_copy(tab_hbm.at[IB[p]], GB[p], GS[p]).wait()
        pltpu.async_copy(GB[p], out_hbm.at[pl.ds(obase + c * CHP, CHP)], SS[p])

    for c in range(DIST):
        fetch(c, c % NBUF, wait=False)

    for c in range(NBUF):
        body(c, c % NBUF)
        fetch(c + DIST, (c + DIST) % NBUF, wait=(c + DIST >= NBUF))

    def main(k, carry):
        for p in range(NBUF):
            c = NBUF * k + p
            body(c, p)
            fetch(c + DIST, (p + DIST) % NBUF, wait=True)
        return carry

    lax.fori_loop(1, 4, main, 0)

    for c in range(4 * NBUF, NCH):
        body(c, c % NBUF)
        if c + DIST < NCH:
            fetch(c + DIST, (c + DIST) % NBUF, wait=True)

    for q in range(NBUF):
        wait_store(q)


def kernel(x, tables, bias):
    xp = jnp.pad(x.astype(jnp.int32), ((0, 0), (0, FP - F))).reshape(B * FP)
    fused = _fuse(tables, bias).reshape(F * V, D)
    foff = jnp.asarray(_FOFF)

    mesh = plsc.VectorSubcoreMesh(core_axis_name="c", subcore_axis_name="s")
    run = pl.kernel(
        _gather_body,
        out_type=jax.ShapeDtypeStruct((B * FP, D), jnp.float32),
        mesh=mesh,
        scratch_types=(
            [pltpu.VMEM((CHP,), jnp.int32)]
            + [pltpu.VMEM((CHP,), jnp.int32) for _ in range(NBUF)]
            + [pltpu.VMEM((CHP,), jnp.int32) for _ in range(NBUF)]
            + [pltpu.VMEM((CHP, D), jnp.float32) for _ in range(NBUF)]
            + [pltpu.SemaphoreType.DMA for _ in range(NBUF)]
            + [pltpu.SemaphoreType.DMA for _ in range(NBUF)]
        ),
    )
    out = run(xp, foff, fused)
    return out.reshape(B, FP, D)[:, :F, :]


# speedup vs baseline: 6.0378x; 6.0378x over previous
import numpy as np
import jax
import jax.numpy as jnp
from jax import lax
from jax.experimental import pallas as pl
from jax.experimental.pallas import tpu as pltpu
from jax.experimental.pallas import tpu_sc as plsc

F = 26
V = 1000
D = 128
B = 4096

NW = 32
ROWS = B * F
RPW = ROWS // NW
CH = 128
NCH = RPW // CH
NBUF = 6
DIST = 4

_FOFF = np.asarray((np.arange(ROWS) % F) * V, dtype=np.int32)


def _fuse_body(tab_ref, bias_ref, out_ref):
    out_ref[...] = tab_ref[...] + bias_ref[...]


def _fuse(tables, bias):
    return pl.pallas_call(
        _fuse_body,
        grid=(F,),
        in_specs=[
            pl.BlockSpec((1, V, D), lambda f: (f, 0, 0)),
            pl.BlockSpec((1, 1, D), lambda f: (f, 0, 0)),
        ],
        out_specs=pl.BlockSpec((1, V, D), lambda f: (f, 0, 0)),
        out_shape=jax.ShapeDtypeStruct((F, V, D), jnp.float32),
    )(tables, bias.reshape(F, 1, D))


def _gather_body(x_hbm, foff_hbm, tab_hbm, out_hbm,
                 xb0, xb1, xb2, xb3, xb4, xb5,
                 fb0, fb1, fb2, fb3, fb4, fb5,
                 gb0, gb1, gb2, gb3, gb4, gb5,
                 gs0, gs1, gs2, gs3, gs4, gs5,
                 ss0, ss1, ss2, ss3, ss4, ss5):
    wid = lax.axis_index("s") * 2 + lax.axis_index("c")
    base = wid * RPW

    XB = (xb0, xb1, xb2, xb3, xb4, xb5)
    FB = (fb0, fb1, fb2, fb3, fb4, fb5)
    GB = (gb0, gb1, gb2, gb3, gb4, gb5)
    GS = (gs0, gs1, gs2, gs3, gs4, gs5)
    SS = (ss0, ss1, ss2, ss3, ss4, ss5)

    def wait_store(q):
        pltpu.make_async_copy(GB[q], out_hbm.at[pl.ds(base, CH)], SS[q]).wait()

    def fetch(c, q, wait):
        if wait:
            wait_store(q)
        rbase = base + c * CH
        pltpu.sync_copy(x_hbm.at[pl.ds(rbase, CH)], XB[q])
        pltpu.sync_copy(foff_hbm.at[pl.ds(rbase, CH)], FB[q])
        for i in range(CH // 16):
            sl = pl.ds(i * 16, 16)
            XB[q][sl] = XB[q][sl] + FB[q][sl]
        pltpu.async_copy(tab_hbm.at[XB[q]], GB[q], GS[q])

    def body(c, p):
        pltpu.make_async_copy(tab_hbm.at[XB[p]], GB[p], GS[p]).wait()
        pltpu.async_copy(GB[p], out_hbm.at[pl.ds(base + c * CH, CH)], SS[p])

    for c in range(DIST):
        fetch(c, c % NBUF, wait=False)

    for c in range(NBUF):
        body(c, c % NBUF)
        fetch(c + DIST, (c + DIST) % NBUF, wait=(c + DIST >= NBUF))

    def main(k, carry):
        for p in range(NBUF):
            c = NBUF * k + p
            body(c, p)
            fetch(c + DIST, (p + DIST) % NBUF, wait=True)
        return carry

    lax.fori_loop(1, 3, main, 0)

    for c in range(3 * NBUF, NCH):
        body(c, c % NBUF)
        if c + DIST < NCH:
            fetch(c + DIST, (c + DIST) % NBUF, wait=True)

    for q in range(NBUF):
        wait_store(q)


def kernel(x, tables, bias):
    x_flat = x.reshape(ROWS).astype(jnp.int32)
    fused = _fuse(tables, bias).reshape(F * V, D)
    foff = jnp.asarray(_FOFF)

    mesh = plsc.VectorSubcoreMesh(core_axis_name="c", subcore_axis_name="s")
    run = pl.kernel(
        _gather_body,
        out_type=jax.ShapeDtypeStruct((ROWS, D), jnp.float32),
        mesh=mesh,
        scratch_types=(
            [pltpu.VMEM((CH,), jnp.int32) for _ in range(NBUF)]
            + [pltpu.VMEM((CH,), jnp.int32) for _ in range(NBUF)]
            + [pltpu.VMEM((CH, D), jnp.float32) for _ in range(NBUF)]
            + [pltpu.SemaphoreType.DMA for _ in range(NBUF)]
            + [pltpu.SemaphoreType.DMA for _ in range(NBUF)]
        ),
    )
    out = run(x_flat, foff, fused)
    return out.reshape(B, F, D)


# speedup vs baseline: 6.1606x; 1.0203x over previous
import numpy as np
import jax
import jax.numpy as jnp
from jax import lax
from jax.experimental import pallas as pl
from jax.experimental.pallas import tpu as pltpu
from jax.experimental.pallas import tpu_sc as plsc

F = 26
V = 1000
D = 128
B = 4096

NW = 32
ROWS = B * F
RPW = ROWS // NW
CH = 128
NCH = RPW // CH
NBUF = 6
DIST = 4

_FOFF = np.asarray((np.arange(ROWS) % F) * V, dtype=np.int32)


def _fuse_body(tab_ref, bias_ref, out_ref):
    out_ref[...] = tab_ref[...] + bias_ref[...]


def _fuse(tables, bias):
    return pl.pallas_call(
        _fuse_body,
        grid=(F // 2,),
        in_specs=[
            pl.BlockSpec((2, V, D), lambda f: (f, 0, 0)),
            pl.BlockSpec((2, 1, D), lambda f: (f, 0, 0)),
        ],
        out_specs=pl.BlockSpec((2, V, D), lambda f: (f, 0, 0)),
        out_shape=jax.ShapeDtypeStruct((F, V, D), jnp.float32),
    )(tables, bias.reshape(F, 1, D))


def _gather_body(x_hbm, foff_hbm, tab_hbm, out_hbm,
                 xb0, xb1, xb2, xb3, xb4, xb5,
                 fb0, fb1, fb2, fb3, fb4, fb5,
                 gb0, gb1, gb2, gb3, gb4, gb5,
                 gs0, gs1, gs2, gs3, gs4, gs5,
                 ss0, ss1, ss2, ss3, ss4, ss5):
    wid = lax.axis_index("s") * 2 + lax.axis_index("c")
    base = wid * RPW

    XB = (xb0, xb1, xb2, xb3, xb4, xb5)
    FB = (fb0, fb1, fb2, fb3, fb4, fb5)
    GB = (gb0, gb1, gb2, gb3, gb4, gb5)
    GS = (gs0, gs1, gs2, gs3, gs4, gs5)
    SS = (ss0, ss1, ss2, ss3, ss4, ss5)

    def wait_store(q):
        pltpu.make_async_copy(GB[q], out_hbm.at[pl.ds(base, CH)], SS[q]).wait()

    def fetch(c, q, wait):
        if wait:
            wait_store(q)
        rbase = base + c * CH
        pltpu.sync_copy(x_hbm.at[pl.ds(rbase, CH)], XB[q])
        pltpu.sync_copy(foff_hbm.at[pl.ds(rbase, CH)], FB[q])
        for i in range(CH // 16):
            sl = pl.ds(i * 16, 16)
            XB[q][sl] = XB[q][sl] + FB[q][sl]
        pltpu.async_copy(tab_hbm.at[XB[q]], GB[q], GS[q])

    def body(c, p):
        pltpu.make_async_copy(tab_hbm.at[XB[p]], GB[p], GS[p]).wait()
        pltpu.async_copy(GB[p], out_hbm.at[pl.ds(base + c * CH, CH)], SS[p])

    for c in range(DIST):
        fetch(c, c % NBUF, wait=False)

    for c in range(NBUF):
        body(c, c % NBUF)
        fetch(c + DIST, (c + DIST) % NBUF, wait=(c + DIST >= NBUF))

    def main(k, carry):
        for p in range(NBUF):
            c = NBUF * k + p
            body(c, p)
            fetch(c + DIST, (p + DIST) % NBUF, wait=True)
        return carry

    lax.fori_loop(1, 3, main, 0)

    for c in range(3 * NBUF, NCH):
        body(c, c % NBUF)
        if c + DIST < NCH:
            fetch(c + DIST, (c + DIST) % NBUF, wait=True)

    for q in range(NBUF):
        wait_store(q)


def kernel(x, tables, bias):
    x_flat = x.reshape(ROWS).astype(jnp.int32)
    fused = _fuse(tables, bias).reshape(F * V, D)
    foff = jnp.asarray(_FOFF)

    mesh = plsc.VectorSubcoreMesh(core_axis_name="c", subcore_axis_name="s")
    run = pl.kernel(
        _gather_body,
        out_type=jax.ShapeDtypeStruct((ROWS, D), jnp.float32),
        mesh=mesh,
        scratch_types=(
            [pltpu.VMEM((CH,), jnp.int32) for _ in range(NBUF)]
            + [pltpu.VMEM((CH,), jnp.int32) for _ in range(NBUF)]
            + [pltpu.VMEM((CH, D), jnp.float32) for _ in range(NBUF)]
            + [pltpu.SemaphoreType.DMA for _ in range(NBUF)]
            + [pltpu.SemaphoreType.DMA for _ in range(NBUF)]
        ),
    )
    out = run(x_flat, foff, fused)
    return out.reshape(B, F, D)
